# baseline (speedup 1.0000x reference)
import functools

import jax
import jax.numpy as jnp
from jax import lax
from jax.experimental import pallas as pl
from jax.experimental.pallas import tpu as pltpu
from jax.experimental.pallas import tpu_sc as plsc

B = 4096
T = 200
D = 64
NUM_CORES = 2
NUM_SUBCORES = 16
NW = NUM_CORES * NUM_SUBCORES
ROWS_PER_W = B // NW
TA = 104
TB = T - TA
NBUF = 4


def _emb_body(ids_hbm, tok_hbm, pos_hbm, out_hbm,
              idx_all, pos_sh, rows_a, rows_b, gsem, osem):
    sid = lax.axis_index("s")
    wid = sid * NUM_CORES + lax.axis_index("c")
    wbase = wid * (ROWS_PER_W * T)

    pltpu.sync_copy(ids_hbm.at[pl.ds(wbase, ROWS_PER_W * T)], idx_all)

    @pl.when(sid == 0)
    def _():
        pltpu.sync_copy(pos_hbm, pos_sh)

    plsc.subcore_barrier()

    def start_gather(p, j):
        off = j * T
        ra = rows_a.at[p]
        rb = rows_b.at[p]
        pltpu.sync_copy(pos_sh.at[pl.ds(0, TA)], ra)
        pltpu.sync_copy(pos_sh.at[pl.ds(TA, TB)], rb)
        pltpu.async_copy(tok_hbm.at[idx_all.at[pl.ds(off, TA)]], ra,
                         gsem.at[p], add=True)
        pltpu.async_copy(tok_hbm.at[idx_all.at[pl.ds(off + TA, TB)]], rb,
                         gsem.at[p], add=True)

    def wait_out(p):
        pltpu.make_async_copy(rows_a.at[p], out_hbm.at[pl.ds(0, TA)],
                              osem.at[p]).wait()
        pltpu.make_async_copy(rows_b.at[p], out_hbm.at[pl.ds(0, TB)],
                              osem.at[p]).wait()

    def finish_row(p, j):
        pltpu.make_async_copy(tok_hbm.at[idx_all.at[pl.ds(0, TA)]],
                              rows_a.at[p], gsem.at[p]).wait()
        pltpu.make_async_copy(tok_hbm.at[idx_all.at[pl.ds(0, TB)]],
                              rows_b.at[p], gsem.at[p]).wait()
        off = wbase + j * T
        pltpu.async_copy(rows_a.at[p], out_hbm.at[pl.ds(off, TA)], osem.at[p])
        pltpu.async_copy(rows_b.at[p], out_hbm.at[pl.ds(off + TA, TB)],
                         osem.at[p])

    start_gather(0, 0)

    def body(j, carry):
        p = lax.rem(j, NBUF)
        pn = lax.rem(j + 1, NBUF)

        @pl.when(j + 1 < ROWS_PER_W)
        def _():
            @pl.when(j >= NBUF - 1)
            def _():
                wait_out(pn)

            start_gather(pn, j + 1)

        finish_row(p, j)
        return carry

    lax.fori_loop(0, ROWS_PER_W, body, 0)

    for p in range(NBUF):
        wait_out(p)


@jax.jit
def _emb(ids_flat, token_emb, pos_emb):
    mesh = plsc.VectorSubcoreMesh(core_axis_name="c", subcore_axis_name="s")
    kern = functools.partial(
        pl.kernel,
        out_type=jax.ShapeDtypeStruct((B * T, D), jnp.float32),
        mesh=mesh,
        scratch_types=[
            pltpu.VMEM((ROWS_PER_W * T,), jnp.int32),
            pltpu.VMEM_SHARED((T, D), jnp.float32),
            pltpu.VMEM((NBUF, TA, D), jnp.float32),
            pltpu.VMEM((NBUF, TB, D), jnp.float32),
            pltpu.SemaphoreType.DMA((NBUF,)),
            pltpu.SemaphoreType.DMA((NBUF,)),
        ],
        compiler_params=pltpu.CompilerParams(use_tc_tiling_on_sc=False),
    )(_emb_body)
    return kern(ids_flat, token_emb, pos_emb)


def kernel(input_ids, token_emb, pos_emb):
    ids_flat = input_ids.astype(jnp.int32).reshape(B * T)
    out = _emb(ids_flat, token_emb, pos_emb)
    return out.reshape(B, T, D)


# speedup vs baseline: 1.0013x; 1.0013x over previous
import functools

import jax
import jax.numpy as jnp
from jax import lax
from jax.experimental import pallas as pl
from jax.experimental.pallas import tpu as pltpu
from jax.experimental.pallas import tpu_sc as plsc

B = 4096
T = 200
D = 64
NUM_CORES = 2
NUM_SUBCORES = 16
NW = NUM_CORES * NUM_SUBCORES
ROWS_PER_W = B // NW
TA = 104
TB = T - TA
NBUF = 4


def _emb_body(ids_hbm, tok_hbm, pos_hbm, out_hbm,
              idx_all, pos_sh, rows_a, rows_b, gsem, osem):
    sid = lax.axis_index("s")
    wid = sid * NUM_CORES + lax.axis_index("c")
    wbase = wid * (ROWS_PER_W * T)

    pltpu.sync_copy(ids_hbm.at[pl.ds(wbase, ROWS_PER_W * T)], idx_all)

    @pl.when(sid == 0)
    def _():
        pltpu.sync_copy(pos_hbm, pos_sh)

    plsc.subcore_barrier()

    def start_gather(p, j):
        off = j * T
        ra = rows_a.at[p]
        rb = rows_b.at[p]
        pltpu.sync_copy(pos_sh.at[pl.ds(0, TA)], ra)
        pltpu.sync_copy(pos_sh.at[pl.ds(TA, TB)], rb)
        pltpu.async_copy(tok_hbm.at[idx_all.at[pl.ds(off, TA)]], ra,
                         gsem.at[p], add=True)
        pltpu.async_copy(tok_hbm.at[idx_all.at[pl.ds(off + TA, TB)]], rb,
                         gsem.at[p], add=True)

    def wait_out(p):
        pltpu.make_async_copy(rows_a.at[p], out_hbm.at[pl.ds(0, TA)],
                              osem.at[p]).wait()
        pltpu.make_async_copy(rows_b.at[p], out_hbm.at[pl.ds(0, TB)],
                              osem.at[p]).wait()

    def finish_row(p, j):
        pltpu.make_async_copy(tok_hbm.at[idx_all.at[pl.ds(0, TA)]],
                              rows_a.at[p], gsem.at[p]).wait()
        pltpu.make_async_copy(tok_hbm.at[idx_all.at[pl.ds(0, TB)]],
                              rows_b.at[p], gsem.at[p]).wait()
        off = wbase + j * T
        pltpu.async_copy(rows_a.at[p], out_hbm.at[pl.ds(off, TA)], osem.at[p])
        pltpu.async_copy(rows_b.at[p], out_hbm.at[pl.ds(off + TA, TB)],
                         osem.at[p])

    start_gather(0, 0)

    def body(j, carry):
        p = lax.rem(j, NBUF)
        pn = lax.rem(j + 1, NBUF)

        @pl.when(j + 1 < ROWS_PER_W)
        def _():
            @pl.when(j >= NBUF - 1)
            def _():
                wait_out(pn)

            start_gather(pn, j + 1)

        finish_row(p, j)
        return carry

    lax.fori_loop(0, ROWS_PER_W, body, 0)

    for p in range(NBUF):
        wait_out(p)


@jax.jit
def _emb(ids_flat, token_emb, pos_emb):
    mesh = plsc.VectorSubcoreMesh(core_axis_name="c", subcore_axis_name="s")
    kern = functools.partial(
        pl.kernel,
        out_type=jax.ShapeDtypeStruct((B * T, D), jnp.float32),
        mesh=mesh,
        scratch_types=[
            pltpu.VMEM((ROWS_PER_W * T,), jnp.int32),
            pltpu.VMEM_SHARED((T, D), jnp.float32),
            pltpu.VMEM((NBUF, TA, D), jnp.float32),
            pltpu.VMEM((NBUF, TB, D), jnp.float32),
            pltpu.SemaphoreType.DMA((NBUF,)),
            pltpu.SemaphoreType.DMA((NBUF,)),
        ],
        compiler_params=pltpu.CompilerParams(use_tc_tiling_on_sc=False,
                                             skip_device_barrier=True),
    )(_emb_body)
    return kern(ids_flat, token_emb, pos_emb)


def kernel(input_ids, token_emb, pos_emb):
    ids_flat = input_ids.astype(jnp.int32).reshape(B * T)
    out = _emb(ids_flat, token_emb, pos_emb)
    return out.reshape(B, T, D)
